# baseline (speedup 1.0000x reference)
import functools

import jax
import jax.numpy as jnp
from jax import lax
from jax.experimental import pallas as pl
from jax.experimental.pallas import tpu as pltpu
from jax.experimental.pallas import tpu_sc as plsc

_NC = 2
_NS = 16
_NW = _NC * _NS
_CHUNK = 128


def _sc_gather(x, idx2d):
    nchunks_total, chunk = idx2d.shape
    value_dim = x.shape[1]
    num_indices = nchunks_total * chunk
    chunks_per_w = nchunks_total // _NW
    b_per_w = chunks_per_w * chunk
    mesh = plsc.VectorSubcoreMesh(core_axis_name="c", subcore_axis_name="s")

    @functools.partial(
        pl.kernel,
        mesh=mesh,
        out_type=jax.ShapeDtypeStruct((num_indices, value_dim), x.dtype),
        scratch_types=[
            pltpu.VMEM((chunks_per_w, chunk), jnp.int32),
            pltpu.VMEM((chunk, value_dim), x.dtype),
            pltpu.SemaphoreType.DMA,
        ],
        compiler_params=pltpu.CompilerParams(use_tc_tiling_on_sc=False),
    )
    def gather_kernel(table_hbm, idx_hbm, out_hbm, idx_v, rows_v, sem):
        wid = lax.axis_index("s") * _NC + lax.axis_index("c")
        base = wid * b_per_w
        pltpu.sync_copy(idx_hbm.at[pl.ds(wid * chunks_per_w, chunks_per_w)],
                        idx_v)

        @pl.loop(0, chunks_per_w)
        def _(j):
            pltpu.async_copy(table_hbm.at[idx_v.at[j]], rows_v, sem).wait()
            pltpu.sync_copy(rows_v, out_hbm.at[pl.ds(base + j * chunk, chunk)])

    return gather_kernel(x, idx2d)


@jax.jit
def kernel(x, indices):
    b, s = indices.shape
    idx2d = indices.reshape(b * s // _CHUNK, _CHUNK).astype(jnp.int32)
    out = _sc_gather(x, idx2d)
    return out.reshape(b, s, x.shape[1])


# speedup vs baseline: 1.0483x; 1.0483x over previous
import functools

import jax
import jax.numpy as jnp
from jax import lax
from jax.experimental import pallas as pl
from jax.experimental.pallas import tpu as pltpu
from jax.experimental.pallas import tpu_sc as plsc

_NC = 2
_NS = 16
_NW = _NC * _NS
_CHUNK = 128
_NBUF = 5


def _sc_gather(x, idx2d):
    nchunks_total, chunk = idx2d.shape
    value_dim = x.shape[1]
    num_indices = nchunks_total * chunk
    chunks_per_w = nchunks_total // _NW
    b_per_w = chunks_per_w * chunk
    mesh = plsc.VectorSubcoreMesh(core_axis_name="c", subcore_axis_name="s")

    @functools.partial(
        pl.kernel,
        mesh=mesh,
        out_type=jax.ShapeDtypeStruct((num_indices, value_dim), x.dtype),
        scratch_types=[
            pltpu.VMEM((chunks_per_w, chunk), jnp.int32),
            *[pltpu.VMEM((chunk, value_dim), x.dtype) for _ in range(_NBUF)],
            *[pltpu.SemaphoreType.DMA for _ in range(2 * _NBUF)],
        ],
        compiler_params=pltpu.CompilerParams(use_tc_tiling_on_sc=False),
    )
    def gather_kernel(table_hbm, idx_hbm, out_hbm, idx_v, *rest):
        bufs = rest[:_NBUF]
        gsems = rest[_NBUF:2 * _NBUF]
        wsems = rest[2 * _NBUF:]
        wid = lax.axis_index("s") * _NC + lax.axis_index("c")
        base = wid * b_per_w
        pltpu.sync_copy(idx_hbm.at[pl.ds(wid * chunks_per_w, chunks_per_w)],
                        idx_v)

        for b in range(_NBUF):
            pltpu.async_copy(table_hbm.at[idx_v.at[b]], bufs[b], gsems[b])

        @pl.loop(0, chunks_per_w, step=_NBUF)
        def _(g):
            for b in range(_NBUF):
                c = g + b
                out_slice = out_hbm.at[pl.ds(base + c * chunk, chunk)]
                pltpu.make_async_copy(
                    table_hbm.at[idx_v.at[c]], bufs[b], gsems[b]).wait()
                pltpu.async_copy(bufs[b], out_slice, wsems[b])
                nxt = c + _NBUF

                @pl.when(nxt < chunks_per_w)
                def _():
                    pltpu.make_async_copy(bufs[b], out_slice, wsems[b]).wait()
                    pltpu.async_copy(
                        table_hbm.at[idx_v.at[nxt]], bufs[b], gsems[b])

        for b in range(_NBUF):
            pltpu.make_async_copy(
                bufs[b], out_hbm.at[pl.ds(base, chunk)], wsems[b]).wait()

    return gather_kernel(x, idx2d)


@jax.jit
def kernel(x, indices):
    b, s = indices.shape
    idx2d = indices.reshape(b * s // _CHUNK, _CHUNK).astype(jnp.int32)
    out = _sc_gather(x, idx2d)
    return out.reshape(b, s, x.shape[1])
